# Initial kernel scaffold; baseline (speedup 1.0000x reference)
import functools

import jax
import jax.numpy as jnp
from jax import lax
from jax.experimental import pallas as pl
from jax.experimental.pallas import tpu as pltpu
from jax.experimental.pallas import tpu_sc as plsc

N_EMBED = 8192
DIM = 32
N_TOK = 16384


TOK_BLOCK = 512


def _argmin_body(x_ref, e_ref, idx_ref):
    xt = x_ref[...]
    e = e_ref[...]
    x2 = jnp.sum(xt * xt, axis=1, keepdims=True)
    e2 = jnp.sum(e * e, axis=1)[None, :]
    prod = lax.dot_general(xt, e, (((1,), (1,)), ((), ())),
                           preferred_element_type=jnp.float32)
    d = (x2 + e2) - 2.0 * prod
    m = jnp.min(d, axis=1, keepdims=True)
    ii = lax.broadcasted_iota(jnp.int32, d.shape, 1)
    idx = jnp.min(jnp.where(d <= m, ii, jnp.int32(2**30)), axis=1)
    idx_ref[...] = idx


def _argmin_indices(flat_x, embed):
    grid = N_TOK // TOK_BLOCK
    return pl.pallas_call(
        _argmin_body,
        grid=(grid,),
        in_specs=[
            pl.BlockSpec((TOK_BLOCK, DIM), lambda i: (i, 0)),
            pl.BlockSpec((N_EMBED, DIM), lambda i: (0, 0)),
        ],
        out_specs=pl.BlockSpec((TOK_BLOCK,), lambda i: (i,)),
        out_shape=jax.ShapeDtypeStruct((N_TOK,), jnp.int32),
    )(flat_x, embed)


NC, NS, LANES = 2, 16, 16
NW = NC * NS
B_PER_W = N_TOK // NW
CHUNK = 128
N_CHUNK = B_PER_W // CHUNK


def _gather_body(table_hbm, idx_hbm, out_hbm, idx_v, rows_v, sem):
    wid = lax.axis_index("s") * NC + lax.axis_index("c")
    pltpu.sync_copy(idx_hbm.at[pl.ds(wid * N_CHUNK, N_CHUNK)], idx_v)
    copies = [
        pltpu.async_copy(table_hbm.at[idx_v.at[j]],
                         rows_v.at[pl.ds(j * CHUNK, CHUNK)], sem)
        for j in range(N_CHUNK)
    ]
    for c in copies:
        c.wait()
    pltpu.sync_copy(rows_v, out_hbm.at[pl.ds(wid * B_PER_W, B_PER_W)])


def _sc_gather(embed, idx):
    mesh = plsc.VectorSubcoreMesh(core_axis_name="c", subcore_axis_name="s")
    f = functools.partial(
        pl.kernel,
        mesh=mesh,
        out_type=jax.ShapeDtypeStruct((N_TOK, DIM), jnp.float32),
        scratch_types=[
            pltpu.VMEM((N_CHUNK, CHUNK), jnp.int32),
            pltpu.VMEM((B_PER_W, DIM), jnp.float32),
            pltpu.SemaphoreType.DMA,
        ],
    )(_gather_body)
    return f(embed, idx.reshape(NW * N_CHUNK, CHUNK))


def kernel(x, embed):
    flat_x = x.reshape(-1, DIM)
    idx = _argmin_indices(flat_x, embed)
    quantized = _sc_gather(embed, idx).reshape(x.shape)
    return x + lax.stop_gradient(quantized - x)


# speedup vs baseline: 3.8295x; 3.8295x over previous
import functools

import jax
import jax.numpy as jnp
from jax import lax
from jax.experimental import pallas as pl
from jax.experimental.pallas import tpu as pltpu
from jax.experimental.pallas import tpu_sc as plsc

N_EMBED = 8192
DIM = 32
N_TOK = 16384


TOK_BLOCK = 512


def _argmin_body(x_ref, e_ref, idx_ref):
    xt = x_ref[...]
    e = e_ref[...]
    x2 = jnp.sum(xt * xt, axis=1, keepdims=True)
    e2 = jnp.sum(e * e, axis=1)[None, :]
    prod = lax.dot_general(xt, e, (((1,), (1,)), ((), ())),
                           preferred_element_type=jnp.float32)
    d = (x2 + e2) - 2.0 * prod
    m = jnp.min(d, axis=1, keepdims=True)
    ii = lax.broadcasted_iota(jnp.int32, d.shape, 1)
    idx = jnp.min(jnp.where(d <= m, ii, jnp.int32(2**30)), axis=1)
    idx_ref[...] = idx


def _argmin_indices(flat_x, embed):
    grid = N_TOK // TOK_BLOCK
    return pl.pallas_call(
        _argmin_body,
        grid=(grid,),
        in_specs=[
            pl.BlockSpec((TOK_BLOCK, DIM), lambda i: (i, 0)),
            pl.BlockSpec((N_EMBED, DIM), lambda i: (0, 0)),
        ],
        out_specs=pl.BlockSpec((TOK_BLOCK,), lambda i: (i,)),
        out_shape=jax.ShapeDtypeStruct((N_TOK,), jnp.int32),
    )(flat_x, embed)


NC, NS, LANES = 2, 16, 16
NW = NC * NS
B_PER_W = N_TOK // NW
CHUNK = 128
N_CHUNK = B_PER_W // CHUNK


def _gather_body(table_hbm, idx_hbm, out_hbm, idx_v, rows_v, sem):
    wid = lax.axis_index("s") * NC + lax.axis_index("c")
    pltpu.sync_copy(idx_hbm.at[pl.ds(wid * N_CHUNK, N_CHUNK)], idx_v)
    copies = [
        pltpu.async_copy(table_hbm.at[idx_v.at[j]],
                         rows_v.at[pl.ds(j * CHUNK, CHUNK)], sem)
        for j in range(N_CHUNK)
    ]
    for c in copies:
        c.wait()
    pltpu.sync_copy(rows_v, out_hbm.at[pl.ds(wid * B_PER_W, B_PER_W)])


def _sc_gather(embed, idx):
    mesh = plsc.VectorSubcoreMesh(core_axis_name="c", subcore_axis_name="s")
    f = functools.partial(
        pl.kernel,
        mesh=mesh,
        out_type=jax.ShapeDtypeStruct((N_TOK, DIM), jnp.float32),
        scratch_types=[
            pltpu.VMEM((N_CHUNK, CHUNK), jnp.int32),
            pltpu.VMEM((B_PER_W, DIM), jnp.float32),
            pltpu.SemaphoreType.DMA,
        ],
        compiler_params=pltpu.CompilerParams(use_tc_tiling_on_sc=False),
    )(_gather_body)
    return f(embed, idx.reshape(NW * N_CHUNK, CHUNK))


def kernel(x, embed):
    flat_x = x.reshape(-1, DIM)
    idx = _argmin_indices(flat_x, embed)
    quantized = _sc_gather(embed, idx).reshape(x.shape)
    return x + lax.stop_gradient(quantized - x)


# speedup vs baseline: 4.0578x; 1.0596x over previous
import functools

import jax
import jax.numpy as jnp
from jax import lax
from jax.experimental import pallas as pl
from jax.experimental.pallas import tpu as pltpu
from jax.experimental.pallas import tpu_sc as plsc

N_EMBED = 8192
DIM = 32
N_TOK = 16384


TOK_BLOCK = 512


def _he2_body(et_ref, he2_ref):
    et = et_ref[...]
    he2_ref[...] = 0.5 * jnp.sum(et * et, axis=0, keepdims=True)


def _argmin_body(x_ref, e_ref, he2_ref, idx_ref):
    xt = x_ref[...]
    hx2 = 0.5 * jnp.sum(xt * xt, axis=1, keepdims=True)
    prod = lax.dot_general(xt, e_ref[...], (((1,), (1,)), ((), ())),
                           preferred_element_type=jnp.float32)
    h = (hx2 + he2_ref[...]) - prod
    m = jnp.min(h, axis=1, keepdims=True)
    ii = lax.broadcasted_iota(jnp.int32, (1, N_EMBED), 1).astype(jnp.float32)
    idx = jnp.min(jnp.where(h <= m, ii, jnp.float32(3e38)), axis=1)
    idx_ref[...] = idx.astype(jnp.int32)


def _argmin_indices(flat_x, embed):
    he2 = pl.pallas_call(
        _he2_body,
        out_shape=jax.ShapeDtypeStruct((1, N_EMBED), jnp.float32),
    )(embed.T)
    grid = N_TOK // TOK_BLOCK
    return pl.pallas_call(
        _argmin_body,
        grid=(grid,),
        in_specs=[
            pl.BlockSpec((TOK_BLOCK, DIM), lambda i: (i, 0)),
            pl.BlockSpec((N_EMBED, DIM), lambda i: (0, 0)),
            pl.BlockSpec((1, N_EMBED), lambda i: (0, 0)),
        ],
        out_specs=pl.BlockSpec((TOK_BLOCK,), lambda i: (i,)),
        out_shape=jax.ShapeDtypeStruct((N_TOK,), jnp.int32),
    )(flat_x, embed, he2)


NC, NS, LANES = 2, 16, 16
NW = NC * NS
B_PER_W = N_TOK // NW
CHUNK = 128
N_CHUNK = B_PER_W // CHUNK


def _gather_body(table_hbm, idx_hbm, out_hbm, idx_v, rows_v, sem):
    wid = lax.axis_index("s") * NC + lax.axis_index("c")
    pltpu.sync_copy(idx_hbm.at[pl.ds(wid * N_CHUNK, N_CHUNK)], idx_v)
    copies = [
        pltpu.async_copy(table_hbm.at[idx_v.at[j]],
                         rows_v.at[pl.ds(j * CHUNK, CHUNK)], sem)
        for j in range(N_CHUNK)
    ]
    for c in copies:
        c.wait()
    pltpu.sync_copy(rows_v, out_hbm.at[pl.ds(wid * B_PER_W, B_PER_W)])


def _sc_gather(embed, idx):
    mesh = plsc.VectorSubcoreMesh(core_axis_name="c", subcore_axis_name="s")
    f = functools.partial(
        pl.kernel,
        mesh=mesh,
        out_type=jax.ShapeDtypeStruct((N_TOK, DIM), jnp.float32),
        scratch_types=[
            pltpu.VMEM((N_CHUNK, CHUNK), jnp.int32),
            pltpu.VMEM((B_PER_W, DIM), jnp.float32),
            pltpu.SemaphoreType.DMA,
        ],
        compiler_params=pltpu.CompilerParams(use_tc_tiling_on_sc=False),
    )(_gather_body)
    return f(embed, idx.reshape(NW * N_CHUNK, CHUNK))


def kernel(x, embed):
    flat_x = x.reshape(-1, DIM)
    idx = _argmin_indices(flat_x, embed)
    quantized = _sc_gather(embed, idx).reshape(x.shape)
    return x + lax.stop_gradient(quantized - x)


# speedup vs baseline: 5.9806x; 1.4738x over previous
import functools

import jax
import jax.numpy as jnp
from jax import lax
from jax.experimental import pallas as pl
from jax.experimental.pallas import tpu as pltpu
from jax.experimental.pallas import tpu_sc as plsc

N_EMBED = 8192
DIM = 32
N_TOK = 16384


TOK_BLOCK = 1024


def _he2_body(et_ref, he2_ref):
    et = et_ref[...]
    he2_ref[...] = 0.5 * jnp.sum(et * et, axis=0, keepdims=True)


def _argmin_body(x_ref, e_ref, he2_ref, idx_ref):
    xt = x_ref[...]
    prod = lax.dot_general(xt, e_ref[...], (((1,), (1,)), ((), ())),
                           preferred_element_type=jnp.float32)
    h = he2_ref[...] - prod
    idx_ref[...] = jnp.argmin(h, axis=1).astype(jnp.int32)


def _argmin_indices(flat_x, embed, he2):
    n = flat_x.shape[0]
    grid = n // TOK_BLOCK
    return pl.pallas_call(
        _argmin_body,
        grid=(grid,),
        in_specs=[
            pl.BlockSpec((TOK_BLOCK, DIM), lambda i: (i, 0)),
            pl.BlockSpec((N_EMBED, DIM), lambda i: (0, 0)),
            pl.BlockSpec((1, N_EMBED), lambda i: (0, 0)),
        ],
        out_specs=pl.BlockSpec((TOK_BLOCK,), lambda i: (i,)),
        out_shape=jax.ShapeDtypeStruct((n,), jnp.int32),
    )(flat_x, embed, he2)


NC, NS, LANES = 2, 16, 16
NW = NC * NS
CHUNK = 128


def _make_gather_body(n_chunk, b_per_w):
    def _gather_body(table_hbm, idx_hbm, out_hbm, idx_v, rows_v, sem):
        wid = lax.axis_index("s") * NC + lax.axis_index("c")
        pltpu.sync_copy(idx_hbm.at[pl.ds(wid * n_chunk, n_chunk)], idx_v)
        copies = [
            pltpu.async_copy(table_hbm.at[idx_v.at[j]],
                             rows_v.at[pl.ds(j * CHUNK, CHUNK)], sem)
            for j in range(n_chunk)
        ]
        for c in copies:
            c.wait()
        pltpu.sync_copy(rows_v, out_hbm.at[pl.ds(wid * b_per_w, b_per_w)])
    return _gather_body


def _sc_gather(embed, idx):
    n = idx.shape[0]
    b_per_w = n // NW
    n_chunk = b_per_w // CHUNK
    mesh = plsc.VectorSubcoreMesh(core_axis_name="c", subcore_axis_name="s")
    f = functools.partial(
        pl.kernel,
        mesh=mesh,
        out_type=jax.ShapeDtypeStruct((n, DIM), jnp.float32),
        scratch_types=[
            pltpu.VMEM((n_chunk, CHUNK), jnp.int32),
            pltpu.VMEM((b_per_w, DIM), jnp.float32),
            pltpu.SemaphoreType.DMA,
        ],
        compiler_params=pltpu.CompilerParams(use_tc_tiling_on_sc=False),
    )(_make_gather_body(n_chunk, b_per_w))
    return f(embed, idx.reshape(NW * n_chunk, CHUNK))


def kernel(x, embed):
    flat_x = x.reshape(-1, DIM)
    he2 = pl.pallas_call(
        _he2_body,
        out_shape=jax.ShapeDtypeStruct((1, N_EMBED), jnp.float32),
    )(embed.T)
    half = N_TOK // 2
    idx0 = _argmin_indices(flat_x[:half], embed, he2)
    q0 = _sc_gather(embed, idx0)
    idx1 = _argmin_indices(flat_x[half:], embed, he2)
    q1 = _sc_gather(embed, idx1)
    return jnp.concatenate([q0, q1], axis=0).reshape(x.shape)


# speedup vs baseline: 6.2708x; 1.0485x over previous
import functools

import jax
import jax.numpy as jnp
from jax import lax
from jax.experimental import pallas as pl
from jax.experimental.pallas import tpu as pltpu
from jax.experimental.pallas import tpu_sc as plsc

N_EMBED = 8192
DIM = 32
N_TOK = 16384


TOK_BLOCK = 1024


def _he2_body(et_ref, he2_ref):
    et = et_ref[...]
    he2_ref[...] = 0.5 * jnp.sum(et * et, axis=0, keepdims=True)


def _argmin_body(x_ref, e_ref, he2_ref, idx_ref):
    xt = x_ref[...]
    prod = lax.dot_general(xt, e_ref[...], (((1,), (1,)), ((), ())),
                           preferred_element_type=jnp.float32)
    h = he2_ref[...] - prod
    idx_ref[...] = jnp.argmin(h, axis=1).astype(jnp.int32)


def _argmin_indices(flat_x, embed, he2):
    n = flat_x.shape[0]
    grid = n // TOK_BLOCK
    return pl.pallas_call(
        _argmin_body,
        grid=(grid,),
        in_specs=[
            pl.BlockSpec((TOK_BLOCK, DIM), lambda i: (i, 0)),
            pl.BlockSpec((N_EMBED, DIM), lambda i: (0, 0)),
            pl.BlockSpec((1, N_EMBED), lambda i: (0, 0)),
        ],
        out_specs=pl.BlockSpec((TOK_BLOCK,), lambda i: (i,)),
        out_shape=jax.ShapeDtypeStruct((n,), jnp.int32),
    )(flat_x, embed, he2)


NC, NS, LANES = 2, 16, 16
NW = NC * NS
CHUNK = 128


def _make_gather_body(n_chunk, b_per_w):
    def _gather_body(table_hbm, idx_hbm, out_hbm, idx_v, rows_v, sem):
        wid = lax.axis_index("s") * NC + lax.axis_index("c")
        pltpu.sync_copy(idx_hbm.at[pl.ds(wid * n_chunk, n_chunk)], idx_v)
        copies = [
            pltpu.async_copy(table_hbm.at[idx_v.at[j]],
                             rows_v.at[pl.ds(j * CHUNK, CHUNK)], sem)
            for j in range(n_chunk)
        ]
        for c in copies:
            c.wait()
        pltpu.sync_copy(rows_v, out_hbm.at[pl.ds(wid * b_per_w, b_per_w)])
    return _gather_body


def _sc_gather(embed, idx):
    n = idx.shape[0]
    b_per_w = n // NW
    n_chunk = b_per_w // CHUNK
    mesh = plsc.VectorSubcoreMesh(core_axis_name="c", subcore_axis_name="s")
    f = functools.partial(
        pl.kernel,
        mesh=mesh,
        out_type=jax.ShapeDtypeStruct((n, DIM), jnp.float32),
        scratch_types=[
            pltpu.VMEM((n_chunk, CHUNK), jnp.int32),
            pltpu.VMEM((b_per_w, DIM), jnp.float32),
            pltpu.SemaphoreType.DMA,
        ],
        compiler_params=pltpu.CompilerParams(use_tc_tiling_on_sc=False),
    )(_make_gather_body(n_chunk, b_per_w))
    return f(embed, idx.reshape(NW * n_chunk, CHUNK))


def kernel(x, embed):
    flat_x = x.reshape(-1, DIM)
    he2 = pl.pallas_call(
        _he2_body,
        out_shape=jax.ShapeDtypeStruct((1, N_EMBED), jnp.float32),
    )(embed.T)
    idx = _argmin_indices(flat_x, embed, he2)
    return _sc_gather(embed, idx).reshape(x.shape)
